# baseline (speedup 1.0000x reference)
import jax
import jax.numpy as jnp
from jax.experimental import pallas as pl
from jax.experimental.pallas import tpu as pltpu


def _fuse_kernel(w1_ref, w2_ref, b1_ref, b2_ref, wt_ref, b_ref):
    wt = jax.lax.dot_general(
        w1_ref[...].astype(jnp.bfloat16),
        w2_ref[...].astype(jnp.bfloat16),
        (((0,), (1,)), ((), ())),
        preferred_element_type=jnp.float32)
    wt_ref[...] = wt.astype(jnp.bfloat16)
    b_ref[...] = b2_ref[...] + jax.lax.dot_general(
        b1_ref[...], w2_ref[...], (((1,), (1,)), ((), ())),
        preferred_element_type=jnp.float32)


def _mlp_kernel(x_ref, wt_ref, b_ref, o_ref):
    acc = jnp.dot(x_ref[...].astype(jnp.bfloat16), wt_ref[...],
                  preferred_element_type=jnp.float32)
    o_ref[...] = (acc + b_ref[...]).astype(o_ref.dtype)


def _pick_tile(n, candidates):
    for c in candidates:
        if n % c == 0:
            return c
    return n


def kernel(x, w1, b1, w2, b2):
    B, D_in = x.shape
    H = w1.shape[0]
    D_out = w2.shape[0]

    b1r = b1.reshape(1, H)
    b2r = b2.reshape(1, D_out)

    tn = D_out // 2 if D_out % 2 == 0 else D_out
    wt, bias = pl.pallas_call(
        _fuse_kernel,
        grid=(D_out // tn,),
        in_specs=[
            pl.BlockSpec((H, D_in), lambda j: (0, 0)),
            pl.BlockSpec((tn, H), lambda j: (j, 0)),
            pl.BlockSpec((1, H), lambda j: (0, 0)),
            pl.BlockSpec((1, tn), lambda j: (0, j)),
        ],
        out_specs=[
            pl.BlockSpec((D_in, tn), lambda j: (0, j)),
            pl.BlockSpec((1, tn), lambda j: (0, j)),
        ],
        out_shape=[
            jax.ShapeDtypeStruct((D_in, D_out), jnp.bfloat16),
            jax.ShapeDtypeStruct((1, D_out), jnp.float32),
        ],
        compiler_params=pltpu.CompilerParams(
            dimension_semantics=("parallel",)),
    )(w1, w2, b1r, b2r)

    tb = _pick_tile(B, (512, 256, 128, 8))
    out = pl.pallas_call(
        _mlp_kernel,
        grid=(B // tb,),
        in_specs=[
            pl.BlockSpec((tb, D_in), lambda i: (i, 0)),
            pl.BlockSpec((D_in, D_out), lambda i: (0, 0)),
            pl.BlockSpec((1, D_out), lambda i: (0, 0)),
        ],
        out_specs=pl.BlockSpec((tb, D_out), lambda i: (i, 0)),
        out_shape=jax.ShapeDtypeStruct((B, D_out), x.dtype),
        compiler_params=pltpu.CompilerParams(
            dimension_semantics=("parallel",)),
    )(x, wt, bias)
    return out


# speedup vs baseline: 1.1337x; 1.1337x over previous
import jax
import jax.numpy as jnp
from jax.experimental import pallas as pl
from jax.experimental.pallas import tpu as pltpu


def _fuse_kernel(w1_ref, w2_ref, b1_ref, b2_ref, wt_ref, b_ref):
    wt = jax.lax.dot_general(
        w1_ref[...].astype(jnp.bfloat16),
        w2_ref[...].astype(jnp.bfloat16),
        (((0,), (1,)), ((), ())),
        preferred_element_type=jnp.float32)
    wt_ref[...] = wt.astype(jnp.bfloat16)
    b_ref[...] = b2_ref[...] + jax.lax.dot_general(
        b1_ref[...], w2_ref[...], (((1,), (1,)), ((), ())),
        preferred_element_type=jnp.float32)


def _mlp_kernel(x_ref, wt_ref, b_ref, o_ref):
    acc = jnp.dot(x_ref[...].astype(jnp.bfloat16), wt_ref[...],
                  preferred_element_type=jnp.float32)
    o_ref[...] = (acc + b_ref[...]).astype(o_ref.dtype)


def _pick_tile(n, candidates):
    for c in candidates:
        if n % c == 0:
            return c
    return n


def kernel(x, w1, b1, w2, b2):
    B, D_in = x.shape
    H = w1.shape[0]
    D_out = w2.shape[0]

    b1r = b1.reshape(1, H)
    b2r = b2.reshape(1, D_out)

    tn = D_out // 2 if D_out % 2 == 0 else D_out
    wt, bias = pl.pallas_call(
        _fuse_kernel,
        grid=(D_out // tn,),
        in_specs=[
            pl.BlockSpec((H, D_in), lambda j: (0, 0)),
            pl.BlockSpec((tn, H), lambda j: (j, 0)),
            pl.BlockSpec((1, H), lambda j: (0, 0)),
            pl.BlockSpec((1, tn), lambda j: (0, j)),
        ],
        out_specs=[
            pl.BlockSpec((D_in, tn), lambda j: (0, j)),
            pl.BlockSpec((1, tn), lambda j: (0, j)),
        ],
        out_shape=[
            jax.ShapeDtypeStruct((D_in, D_out), jnp.bfloat16),
            jax.ShapeDtypeStruct((1, D_out), jnp.float32),
        ],
        compiler_params=pltpu.CompilerParams(
            dimension_semantics=("parallel",)),
    )(w1, w2, b1r, b2r)

    tb = _pick_tile(B, (2048, 1024, 512, 256, 128, 8))
    out = pl.pallas_call(
        _mlp_kernel,
        grid=(B // tb,),
        in_specs=[
            pl.BlockSpec((tb, D_in), lambda i: (i, 0)),
            pl.BlockSpec((D_in, D_out), lambda i: (0, 0)),
            pl.BlockSpec((1, D_out), lambda i: (0, 0)),
        ],
        out_specs=pl.BlockSpec((tb, D_out), lambda i: (i, 0)),
        out_shape=jax.ShapeDtypeStruct((B, D_out), x.dtype),
        compiler_params=pltpu.CompilerParams(
            dimension_semantics=("parallel",)),
    )(x, wt, bias)
    return out
